# baseline (speedup 1.0000x reference)
import jax
import jax.numpy as jnp
from jax import lax
from jax.experimental import pallas as pl
from jax.experimental.pallas import tpu as pltpu
from jax.experimental.pallas import tpu_sc as plsc

N = 10000
M = 160000
HID = 256
H = 16
D = 16
HH = 128

NC = 2
NS = 16
NW = NC * NS

G_PER_W = M // NW
G_CH = 128
G_FULL = G_PER_W // G_CH
G_REM = G_PER_W - G_FULL * G_CH

S_PER_T = M // NS
S_FULL = S_PER_T // G_CH
S_REM = S_PER_T - S_FULL * G_CH

ROWS_PER_T = N // NS
PW = 144

R1 = 400
R2 = 640

_BF = jnp.bfloat16
_F = jnp.float32
_U = jnp.uint32


def _rne_hi(x):
    u = lax.bitcast_convert_type(x, _U)
    r = u + jnp.uint32(0x7FFF) + ((u >> jnp.uint32(16)) & jnp.uint32(1))
    return r & jnp.uint32(0xFFFF0000)


def _pack(a, b):
    return lax.bitcast_convert_type(_rne_hi(a) | (_rne_hi(b) >> jnp.uint32(16)), _F)


def _unpack(x):
    u = lax.bitcast_convert_type(x, _U)
    a = lax.bitcast_convert_type(u & jnp.uint32(0xFFFF0000), _F)
    b = lax.bitcast_convert_type(u << jnp.uint32(16), _F)
    return a, b


def _proj_body(xq_ref, xk_ref, wqa, wqb, wka, wkb, wva, wvb,
               q_ref, k_ref, v_ref):
    dn = (((1,), (1,)), ((), ()))
    xq = xq_ref[...]
    xk = xk_ref[...]
    q_ref[...] = _pack(lax.dot_general(xq, wqa[...], dn),
                       lax.dot_general(xq, wqb[...], dn))
    k_ref[...] = _pack(lax.dot_general(xk, wka[...], dn),
                       lax.dot_general(xk, wkb[...], dn))
    v_ref[...] = _pack(lax.dot_general(xk, wva[...], dn),
                       lax.dot_general(xk, wvb[...], dn))


def _proj_call(xq, xk, *ws):
    grid = (N // R1,)
    row = pl.BlockSpec((R1, HID), lambda i: (i, 0))
    half = pl.BlockSpec((R1, HH), lambda i: (i, 0))
    wfull = pl.BlockSpec((HH, HID), lambda i: (0, 0))
    return pl.pallas_call(
        _proj_body,
        grid=grid,
        in_specs=[row, row] + [wfull] * 6,
        out_specs=[half] * 3,
        out_shape=[jax.ShapeDtypeStruct((N, HH), _F)] * 3,
    )(xq, xk, *ws)


def _gather_body(ta, tb, tc_, qi, ki, oa, ob, oc,
                 qi_t, ki_t, ra, rb, rc, sem_g, sem_w):
    c = lax.axis_index("c")
    s = lax.axis_index("s")
    w = s * NC + c
    base = w * G_PER_W

    pltpu.sync_copy(qi.at[pl.ds(base, G_PER_W)], qi_t)
    pltpu.sync_copy(ki.at[pl.ds(base, G_PER_W)], ki_t)

    tabs = (ta, tb, tc_)
    outs = (oa, ob, oc)
    rows = (ra, rb, rc)
    idx_for = (0, 1, 1)

    def body(i, _):
        e0 = base + i * G_CH
        o = i * G_CH
        sl = (qi_t.at[pl.ds(o, G_CH)], ki_t.at[pl.ds(o, G_CH)])
        cps = [pltpu.async_copy(t.at[sl[ix]], r, sem_g)
               for t, r, ix in zip(tabs, rows, idx_for)]
        for cp in cps:
            cp.wait()
        wps = [pltpu.async_copy(r, ot.at[pl.ds(e0, G_CH)], sem_w)
               for r, ot in zip(rows, outs)]
        for wp in wps:
            wp.wait()
        return _

    lax.fori_loop(0, G_FULL, body, 0)

    e0 = base + G_FULL * G_CH
    o = G_FULL * G_CH
    sl = (qi_t.at[pl.ds(o, G_REM)], ki_t.at[pl.ds(o, G_REM)])
    cps = [pltpu.async_copy(t.at[sl[ix]], r.at[pl.ds(0, G_REM)], sem_g)
           for t, r, ix in zip(tabs, rows, idx_for)]
    for cp in cps:
        cp.wait()
    for r, ot in zip(rows, outs):
        pltpu.sync_copy(r.at[pl.ds(0, G_REM)], ot.at[pl.ds(e0, G_REM)])


def _gather_call(q, k, v, qi, ki):
    mesh = plsc.VectorSubcoreMesh(core_axis_name="c", subcore_axis_name="s")
    f = pl.kernel(
        _gather_body,
        mesh=mesh,
        compiler_params=pltpu.CompilerParams(use_tc_tiling_on_sc=False),
        out_type=[jax.ShapeDtypeStruct((M, HH), _F)] * 3,
        scratch_types=[
            pltpu.VMEM((G_PER_W,), jnp.int32),
            pltpu.VMEM((G_PER_W,), jnp.int32),
            pltpu.VMEM((G_CH, HH), _F),
            pltpu.VMEM((G_CH, HH), _F),
            pltpu.VMEM((G_CH, HH), _F),
            pltpu.SemaphoreType.DMA,
            pltpu.SemaphoreType.DMA,
        ],
    )
    return f(q, k, v, qi, ki)


def _edge_body(qg_ref, kg_ref, vg_ref, ir_ref,
               wbma, wbmb, wbaa, wbab, s8_ref, st8_ref, sel_ref, inv_ref,
               logit_ref, ex_ref, numa_ref, numb_ref):
    dnt = (((1,), (1,)), ((), ()))
    dn = (((1,), (0,)), ((), ()))
    ir = ir_ref[...]
    bma = lax.dot_general(ir, wbma[...], dnt)
    bmb = lax.dot_general(ir, wbmb[...], dnt)
    baa = lax.dot_general(ir, wbaa[...], dnt)
    bab = lax.dot_general(ir, wbab[...], dnt)
    qga, qgb = _unpack(qg_ref[...])
    kga, kgb = _unpack(kg_ref[...])
    vga, vgb = _unpack(vg_ref[...])
    ta = kga * (1.0 + bma) + baa
    tb = kgb * (1.0 + bmb) + bab
    pa = qga * ta
    pb = qgb * tb
    s8 = s8_ref[...]
    inv = inv_ref[0, 0]
    la = lax.dot_general(pa, s8, dn) * inv
    lb = lax.dot_general(pb, s8, dn) * inv
    logit_ref[...] = jnp.concatenate([la, lb], axis=1)
    exa = jnp.exp(la)
    exb = jnp.exp(lb)
    sel = sel_ref[...]
    ex_ref[...] = lax.dot_general(jnp.concatenate([exa, exb], axis=1), sel, dn)
    st8 = st8_ref[...]
    numa_ref[...] = lax.dot_general(exa, st8, dn) * vga
    numb_ref[...] = lax.dot_general(exb, st8, dn) * vgb


def _edge_call(qg, kg, vg, ir, wbma, wbmb, wbaa, wbab, s8, st8, sel, inv):
    grid = (M // R2,)
    half = pl.BlockSpec((R2, HH), lambda i: (i, 0))
    wfull = pl.BlockSpec((HH, HID), lambda i: (0, 0))
    return pl.pallas_call(
        _edge_body,
        grid=grid,
        in_specs=[half] * 3
        + [pl.BlockSpec((R2, HID), lambda i: (i, 0))]
        + [wfull] * 4
        + [pl.BlockSpec((HH, 8), lambda i: (0, 0)),
           pl.BlockSpec((8, HH), lambda i: (0, 0)),
           pl.BlockSpec((H, HH), lambda i: (0, 0)),
           pl.BlockSpec(memory_space=pltpu.SMEM)],
        out_specs=[pl.BlockSpec((R2, H), lambda i: (i, 0)),
                   half, half, half],
        out_shape=[jax.ShapeDtypeStruct((M, H), _F),
                   jax.ShapeDtypeStruct((M, HH), _F),
                   jax.ShapeDtypeStruct((M, HH), _F),
                   jax.ShapeDtypeStruct((M, HH), _F)],
    )(qg, kg, vg, ir, wbma, wbmb, wbaa, wbab, s8, st8, sel, inv)


def _scatter_body(numa, numb, ex, qi, z128, z16, out_o, accum,
                  idx_a, pay_a, idx_b, pay_b, idx_r, pay_r, sem_f):
    c = lax.axis_index("c")
    s = lax.axis_index("s")
    r0 = s * ROWS_PER_T
    col0 = c * PW
    exc0 = c * 8

    pltpu.sync_copy(z128.at[pl.ds(r0, ROWS_PER_T)],
                    accum.at[pl.ds(r0, ROWS_PER_T), pl.ds(0, HH)])
    pltpu.sync_copy(z16.at[pl.ds(r0, ROWS_PER_T)],
                    accum.at[pl.ds(r0, ROWS_PER_T), pl.ds(HH, 16)])
    plsc.subcore_barrier()

    def make_loop(num_ref):
        def body(i, _):
            e0 = s * S_PER_T + 2 * i * G_CH
            e1 = e0 + G_CH
            fa = [pltpu.async_copy(qi.at[pl.ds(e0, G_CH)], idx_a, sem_f),
                  pltpu.async_copy(num_ref.at[pl.ds(e0, G_CH)],
                                   pay_a.at[pl.ds(0, G_CH), pl.ds(0, HH)], sem_f),
                  pltpu.async_copy(ex.at[pl.ds(e0, G_CH), pl.ds(exc0, 8)],
                                   pay_a.at[pl.ds(0, G_CH), pl.ds(HH, 8)], sem_f)]
            fb = [pltpu.async_copy(qi.at[pl.ds(e1, G_CH)], idx_b, sem_f),
                  pltpu.async_copy(num_ref.at[pl.ds(e1, G_CH)],
                                   pay_b.at[pl.ds(0, G_CH), pl.ds(0, HH)], sem_f),
                  pltpu.async_copy(ex.at[pl.ds(e1, G_CH), pl.ds(exc0, 8)],
                                   pay_b.at[pl.ds(0, G_CH), pl.ds(HH, 8)], sem_f)]
            for f_ in fa:
                f_.wait()
            pltpu.sync_copy(pay_a, accum.at[idx_a], add=True)
            for f_ in fb:
                f_.wait()
            pltpu.sync_copy(pay_b, accum.at[idx_b], add=True)
            return _

        lax.fori_loop(0, S_FULL // 2, body, 0)
        e0 = s * S_PER_T + S_FULL * G_CH
        pltpu.sync_copy(qi.at[pl.ds(e0, S_REM)], idx_r)
        pltpu.sync_copy(num_ref.at[pl.ds(e0, S_REM)],
                        pay_r.at[pl.ds(0, S_REM), pl.ds(0, HH)])
        pltpu.sync_copy(ex.at[pl.ds(e0, S_REM), pl.ds(exc0, 8)],
                        pay_r.at[pl.ds(0, S_REM), pl.ds(HH, 8)])
        pltpu.sync_copy(pay_r, accum.at[idx_r], add=True)

    @pl.when(c == 0)
    def _():
        make_loop(numa)

    @pl.when(c != 0)
    def _():
        make_loop(numb)

    plsc.subcore_barrier()
    pltpu.sync_copy(accum.at[pl.ds(r0, ROWS_PER_T)],
                    out_o.at[pl.ds(r0, ROWS_PER_T), pl.ds(col0, PW)])


def _scatter_call(numa, numb, ex, qi, z128, z16):
    mesh = plsc.VectorSubcoreMesh(core_axis_name="c", subcore_axis_name="s")
    f = pl.kernel(
        _scatter_body,
        mesh=mesh,
        compiler_params=pltpu.CompilerParams(use_tc_tiling_on_sc=False),
        out_type=jax.ShapeDtypeStruct((N, 2 * PW), _F),
        scratch_types=[
            pltpu.VMEM_SHARED((N, PW), _F),
            pltpu.VMEM((G_CH,), jnp.int32),
            pltpu.VMEM((G_CH, PW), _F),
            pltpu.VMEM((G_CH,), jnp.int32),
            pltpu.VMEM((G_CH, PW), _F),
            pltpu.VMEM((S_REM,), jnp.int32),
            pltpu.VMEM((S_REM, PW), _F),
            pltpu.SemaphoreType.DMA,
        ],
    )
    return f(numa, numb, ex, qi, z128, z16)


def _final_body(acc_ref, st8_ref, wo_ref, res_ref):
    a = acc_ref[...]
    dn = (((1,), (0,)), ((), ()))
    st8 = st8_ref[...]
    den_a = lax.dot_general(a[:, 128:136], st8, dn) + 1e-16
    den_b = lax.dot_general(a[:, 272:280], st8, dn) + 1e-16
    r = jnp.concatenate([a[:, 0:128] / den_a, a[:, 144:272] / den_b], axis=1)
    res_ref[...] = lax.dot_general(r, wo_ref[...], (((1,), (1,)), ((), ())))


def _final_call(acc, st8, wo):
    grid = (N // R1,)
    return pl.pallas_call(
        _final_body,
        grid=grid,
        in_specs=[pl.BlockSpec((R1, 2 * PW), lambda i: (i, 0)),
                  pl.BlockSpec((8, HH), lambda i: (0, 0)),
                  pl.BlockSpec((HID, HID), lambda i: (0, 0))],
        out_specs=pl.BlockSpec((R1, HID), lambda i: (i, 0)),
        out_shape=jax.ShapeDtypeStruct((N, HID), _F),
    )(acc, st8, wo)


def _head_major_rows(w):
    return w.reshape(D, H, HID).transpose(1, 0, 2).reshape(H * D, HID)


def kernel(query, key, query_index, key_index, interaction_repr,
           Wq, Wkv, Wb, Wo, normalizer):
    wq = _head_major_rows(Wq)
    wk = _head_major_rows(Wkv[:HID])
    wv = _head_major_rows(Wkv[HID:])
    wbm = _head_major_rows(Wb[:HID])
    wba = _head_major_rows(Wb[HID:])
    wo = Wo.reshape(HID, D, H).transpose(0, 2, 1).reshape(HID, HID)

    inv = (1.0 / jnp.clip(normalizer, 1.0, float(D))).reshape(1, 1)
    s8 = jnp.repeat(jnp.eye(8, dtype=_F), D, axis=0)
    st8 = jnp.repeat(jnp.eye(8, dtype=_F), D, axis=1)
    sel = jnp.eye(H, HH, dtype=_F)
    z128 = jnp.zeros((N, HH), _F)
    z16 = jnp.zeros((N, 16), _F)

    q, k, v = _proj_call(
        query, key, wq[:HH], wq[HH:], wk[:HH], wk[HH:], wv[:HH], wv[HH:])
    qg, kg, vg = _gather_call(q, k, v, query_index, key_index)
    logits, ex, numa, numb = _edge_call(
        qg, kg, vg, interaction_repr,
        wbm[:HH], wbm[HH:], wba[:HH], wba[HH:], s8, st8, sel, inv)
    acc = _scatter_call(numa, numb, ex, query_index, z128, z16)
    result = _final_call(acc, st8, wo)
    return (result, logits)


# speedup vs baseline: 1.0231x; 1.0231x over previous
import jax
import jax.numpy as jnp
from jax import lax
from jax.experimental import pallas as pl
from jax.experimental.pallas import tpu as pltpu
from jax.experimental.pallas import tpu_sc as plsc

N = 10000
M = 160000
HID = 256
H = 16
D = 16
HH = 128

NC = 2
NS = 16
NW = NC * NS

G_PER_W = M // NW
G_CH = 128
G_FULL = G_PER_W // G_CH
G_REM = G_PER_W - G_FULL * G_CH

S_PER_T = M // NS
S_FULL = S_PER_T // G_CH
S_REM = S_PER_T - S_FULL * G_CH

ROWS_PER_T = N // NS
PW = 144

R1 = 400
R2 = 640

_BF = jnp.bfloat16
_F = jnp.float32
_U = jnp.uint32


def _rne_hi(x):
    u = lax.bitcast_convert_type(x, _U)
    r = u + jnp.uint32(0x7FFF) + ((u >> jnp.uint32(16)) & jnp.uint32(1))
    return r & jnp.uint32(0xFFFF0000)


def _pack(a, b):
    return lax.bitcast_convert_type(_rne_hi(a) | (_rne_hi(b) >> jnp.uint32(16)), _F)


def _unpack(x):
    u = lax.bitcast_convert_type(x, _U)
    a = lax.bitcast_convert_type(u & jnp.uint32(0xFFFF0000), _F)
    b = lax.bitcast_convert_type(u << jnp.uint32(16), _F)
    return a, b


def _proj_body(xq_ref, xk_ref, wqa, wqb, wka, wkb, wva, wvb,
               q_ref, k_ref, v_ref):
    dn = (((1,), (1,)), ((), ()))
    xq = xq_ref[...]
    xk = xk_ref[...]
    q_ref[...] = _pack(lax.dot_general(xq, wqa[...], dn),
                       lax.dot_general(xq, wqb[...], dn))
    k_ref[...] = _pack(lax.dot_general(xk, wka[...], dn),
                       lax.dot_general(xk, wkb[...], dn))
    v_ref[...] = _pack(lax.dot_general(xk, wva[...], dn),
                       lax.dot_general(xk, wvb[...], dn))


def _proj_call(xq, xk, *ws):
    grid = (N // R1,)
    row = pl.BlockSpec((R1, HID), lambda i: (i, 0))
    half = pl.BlockSpec((R1, HH), lambda i: (i, 0))
    wfull = pl.BlockSpec((HH, HID), lambda i: (0, 0))
    return pl.pallas_call(
        _proj_body,
        grid=grid,
        in_specs=[row, row] + [wfull] * 6,
        out_specs=[half] * 3,
        out_shape=[jax.ShapeDtypeStruct((N, HH), _F)] * 3,
    )(xq, xk, *ws)


def _gather_body(ta, tb, tc_, qi, ki, oa, ob, oc,
                 qi_t, ki_t, ra, rb, rc, sem_g, sem_w):
    c = lax.axis_index("c")
    s = lax.axis_index("s")
    w = s * NC + c
    base = w * G_PER_W

    pltpu.sync_copy(qi.at[pl.ds(base, G_PER_W)], qi_t)
    pltpu.sync_copy(ki.at[pl.ds(base, G_PER_W)], ki_t)

    tabs = (ta, tb, tc_)
    outs = (oa, ob, oc)
    rows = (ra, rb, rc)
    idx_for = (0, 1, 1)

    def body(i, _):
        e0 = base + i * G_CH
        o = i * G_CH
        sl = (qi_t.at[pl.ds(o, G_CH)], ki_t.at[pl.ds(o, G_CH)])
        cps = [pltpu.async_copy(t.at[sl[ix]], r, sem_g)
               for t, r, ix in zip(tabs, rows, idx_for)]
        for cp in cps:
            cp.wait()
        wps = [pltpu.async_copy(r, ot.at[pl.ds(e0, G_CH)], sem_w)
               for r, ot in zip(rows, outs)]
        for wp in wps:
            wp.wait()
        return _

    lax.fori_loop(0, G_FULL, body, 0)

    e0 = base + G_FULL * G_CH
    o = G_FULL * G_CH
    sl = (qi_t.at[pl.ds(o, G_REM)], ki_t.at[pl.ds(o, G_REM)])
    cps = [pltpu.async_copy(t.at[sl[ix]], r.at[pl.ds(0, G_REM)], sem_g)
           for t, r, ix in zip(tabs, rows, idx_for)]
    for cp in cps:
        cp.wait()
    for r, ot in zip(rows, outs):
        pltpu.sync_copy(r.at[pl.ds(0, G_REM)], ot.at[pl.ds(e0, G_REM)])


def _gather_call(q, k, v, qi, ki):
    mesh = plsc.VectorSubcoreMesh(core_axis_name="c", subcore_axis_name="s")
    f = pl.kernel(
        _gather_body,
        mesh=mesh,
        compiler_params=pltpu.CompilerParams(use_tc_tiling_on_sc=False),
        out_type=[jax.ShapeDtypeStruct((M, HH), _F)] * 3,
        scratch_types=[
            pltpu.VMEM((G_PER_W,), jnp.int32),
            pltpu.VMEM((G_PER_W,), jnp.int32),
            pltpu.VMEM((G_CH, HH), _F),
            pltpu.VMEM((G_CH, HH), _F),
            pltpu.VMEM((G_CH, HH), _F),
            pltpu.SemaphoreType.DMA,
            pltpu.SemaphoreType.DMA,
        ],
    )
    return f(q, k, v, qi, ki)


def _edge_body(qg_ref, kg_ref, vg_ref, ir_ref,
               wbma, wbmb, wbaa, wbab, s8_ref, st8_ref, inv_ref,
               logit_ref, ex_ref, numa_ref, numb_ref):
    dnt = (((1,), (1,)), ((), ()))
    dn = (((1,), (0,)), ((), ()))
    ir = ir_ref[...]
    bma = lax.dot_general(ir, wbma[...], dnt)
    bmb = lax.dot_general(ir, wbmb[...], dnt)
    baa = lax.dot_general(ir, wbaa[...], dnt)
    bab = lax.dot_general(ir, wbab[...], dnt)
    qga, qgb = _unpack(qg_ref[...])
    kga, kgb = _unpack(kg_ref[...])
    vga, vgb = _unpack(vg_ref[...])
    ta = kga * (1.0 + bma) + baa
    tb = kgb * (1.0 + bmb) + bab
    pa = qga * ta
    pb = qgb * tb
    s8 = s8_ref[...]
    inv = inv_ref[0, 0]
    la = lax.dot_general(pa, s8, dn) * inv
    lb = lax.dot_general(pb, s8, dn) * inv
    logit_ref[...] = jnp.concatenate([la, lb], axis=1)
    exa = jnp.exp(la)
    exb = jnp.exp(lb)
    z = jnp.zeros((la.shape[0], HH - 2 * 8), _F)
    ex_ref[...] = jnp.concatenate([exa, exb, z], axis=1)
    st8 = st8_ref[...]
    numa_ref[...] = lax.dot_general(exa, st8, dn) * vga
    numb_ref[...] = lax.dot_general(exb, st8, dn) * vgb


def _edge_call(qg, kg, vg, ir, wbma, wbmb, wbaa, wbab, s8, st8, inv):
    grid = (M // R2,)
    half = pl.BlockSpec((R2, HH), lambda i: (i, 0))
    wfull = pl.BlockSpec((HH, HID), lambda i: (0, 0))
    return pl.pallas_call(
        _edge_body,
        grid=grid,
        in_specs=[half] * 3
        + [pl.BlockSpec((R2, HID), lambda i: (i, 0))]
        + [wfull] * 4
        + [pl.BlockSpec((HH, 8), lambda i: (0, 0)),
           pl.BlockSpec((8, HH), lambda i: (0, 0)),
           pl.BlockSpec(memory_space=pltpu.SMEM)],
        out_specs=[pl.BlockSpec((R2, H), lambda i: (i, 0)),
                   half, half, half],
        out_shape=[jax.ShapeDtypeStruct((M, H), _F),
                   jax.ShapeDtypeStruct((M, HH), _F),
                   jax.ShapeDtypeStruct((M, HH), _F),
                   jax.ShapeDtypeStruct((M, HH), _F)],
    )(qg, kg, vg, ir, wbma, wbmb, wbaa, wbab, s8, st8, inv)


def _scatter_body(numa, numb, ex, qi, z128, z16, out_o, accum,
                  idx_a, pay_a, idx_b, pay_b, idx_r, pay_r, sem_f):
    c = lax.axis_index("c")
    s = lax.axis_index("s")
    r0 = s * ROWS_PER_T
    col0 = c * PW
    exc0 = c * 8

    pltpu.sync_copy(z128.at[pl.ds(r0, ROWS_PER_T)],
                    accum.at[pl.ds(r0, ROWS_PER_T), pl.ds(0, HH)])
    pltpu.sync_copy(z16.at[pl.ds(r0, ROWS_PER_T)],
                    accum.at[pl.ds(r0, ROWS_PER_T), pl.ds(HH, 16)])
    plsc.subcore_barrier()

    def make_loop(num_ref):
        def body(i, _):
            e0 = s * S_PER_T + 2 * i * G_CH
            e1 = e0 + G_CH
            fa = [pltpu.async_copy(qi.at[pl.ds(e0, G_CH)], idx_a, sem_f),
                  pltpu.async_copy(num_ref.at[pl.ds(e0, G_CH)],
                                   pay_a.at[pl.ds(0, G_CH), pl.ds(0, HH)], sem_f),
                  pltpu.async_copy(ex.at[pl.ds(e0, G_CH), pl.ds(exc0, 8)],
                                   pay_a.at[pl.ds(0, G_CH), pl.ds(HH, 8)], sem_f)]
            fb = [pltpu.async_copy(qi.at[pl.ds(e1, G_CH)], idx_b, sem_f),
                  pltpu.async_copy(num_ref.at[pl.ds(e1, G_CH)],
                                   pay_b.at[pl.ds(0, G_CH), pl.ds(0, HH)], sem_f),
                  pltpu.async_copy(ex.at[pl.ds(e1, G_CH), pl.ds(exc0, 8)],
                                   pay_b.at[pl.ds(0, G_CH), pl.ds(HH, 8)], sem_f)]
            for f_ in fa:
                f_.wait()
            pltpu.sync_copy(pay_a, accum.at[idx_a], add=True)
            for f_ in fb:
                f_.wait()
            pltpu.sync_copy(pay_b, accum.at[idx_b], add=True)
            return _

        lax.fori_loop(0, S_FULL // 2, body, 0)
        e0 = s * S_PER_T + S_FULL * G_CH
        pltpu.sync_copy(qi.at[pl.ds(e0, S_REM)], idx_r)
        pltpu.sync_copy(num_ref.at[pl.ds(e0, S_REM)],
                        pay_r.at[pl.ds(0, S_REM), pl.ds(0, HH)])
        pltpu.sync_copy(ex.at[pl.ds(e0, S_REM), pl.ds(exc0, 8)],
                        pay_r.at[pl.ds(0, S_REM), pl.ds(HH, 8)])
        pltpu.sync_copy(pay_r, accum.at[idx_r], add=True)

    @pl.when(c == 0)
    def _():
        make_loop(numa)

    @pl.when(c != 0)
    def _():
        make_loop(numb)

    plsc.subcore_barrier()
    pltpu.sync_copy(accum.at[pl.ds(r0, ROWS_PER_T)],
                    out_o.at[pl.ds(r0, ROWS_PER_T), pl.ds(col0, PW)])


def _scatter_call(numa, numb, ex, qi, z128, z16):
    mesh = plsc.VectorSubcoreMesh(core_axis_name="c", subcore_axis_name="s")
    f = pl.kernel(
        _scatter_body,
        mesh=mesh,
        compiler_params=pltpu.CompilerParams(use_tc_tiling_on_sc=False),
        out_type=jax.ShapeDtypeStruct((N, 2 * PW), _F),
        scratch_types=[
            pltpu.VMEM_SHARED((N, PW), _F),
            pltpu.VMEM((G_CH,), jnp.int32),
            pltpu.VMEM((G_CH, PW), _F),
            pltpu.VMEM((G_CH,), jnp.int32),
            pltpu.VMEM((G_CH, PW), _F),
            pltpu.VMEM((S_REM,), jnp.int32),
            pltpu.VMEM((S_REM, PW), _F),
            pltpu.SemaphoreType.DMA,
        ],
    )
    return f(numa, numb, ex, qi, z128, z16)


def _final_body(acc_ref, st8_ref, wo_ref, res_ref):
    a = acc_ref[...]
    dn = (((1,), (0,)), ((), ()))
    st8 = st8_ref[...]
    den_a = lax.dot_general(a[:, 128:136], st8, dn) + 1e-16
    den_b = lax.dot_general(a[:, 272:280], st8, dn) + 1e-16
    r = jnp.concatenate([a[:, 0:128] / den_a, a[:, 144:272] / den_b], axis=1)
    res_ref[...] = lax.dot_general(r, wo_ref[...], (((1,), (1,)), ((), ())))


def _final_call(acc, st8, wo):
    grid = (N // R1,)
    return pl.pallas_call(
        _final_body,
        grid=grid,
        in_specs=[pl.BlockSpec((R1, 2 * PW), lambda i: (i, 0)),
                  pl.BlockSpec((8, HH), lambda i: (0, 0)),
                  pl.BlockSpec((HID, HID), lambda i: (0, 0))],
        out_specs=pl.BlockSpec((R1, HID), lambda i: (i, 0)),
        out_shape=jax.ShapeDtypeStruct((N, HID), _F),
    )(acc, st8, wo)


def _head_major_rows(w):
    return w.reshape(D, H, HID).transpose(1, 0, 2).reshape(H * D, HID)


def kernel(query, key, query_index, key_index, interaction_repr,
           Wq, Wkv, Wb, Wo, normalizer):
    wq = _head_major_rows(Wq)
    wk = _head_major_rows(Wkv[:HID])
    wv = _head_major_rows(Wkv[HID:])
    wbm = _head_major_rows(Wb[:HID])
    wba = _head_major_rows(Wb[HID:])
    wo = Wo.reshape(HID, D, H).transpose(0, 2, 1).reshape(HID, HID)

    inv = (1.0 / jnp.clip(normalizer, 1.0, float(D))).reshape(1, 1)
    s8 = jnp.repeat(jnp.eye(8, dtype=_F), D, axis=0)
    st8 = jnp.repeat(jnp.eye(8, dtype=_F), D, axis=1)
    z128 = jnp.zeros((N, HH), _F)
    z16 = jnp.zeros((N, 16), _F)

    q, k, v = _proj_call(
        query, key, wq[:HH], wq[HH:], wk[:HH], wk[HH:], wv[:HH], wv[HH:])
    qg, kg, vg = _gather_call(q, k, v, query_index, key_index)
    logits, ex, numa, numb = _edge_call(
        qg, kg, vg, interaction_repr,
        wbm[:HH], wbm[HH:], wba[:HH], wba[HH:], s8, st8, inv)
    acc = _scatter_call(numa, numb, ex, query_index, z128, z16)
    result = _final_call(acc, st8, wo)
    return (result, logits)


# speedup vs baseline: 1.0735x; 1.0493x over previous
import jax
import jax.numpy as jnp
from jax import lax
from jax.experimental import pallas as pl
from jax.experimental.pallas import tpu as pltpu
from jax.experimental.pallas import tpu_sc as plsc

N = 10000
M = 160000
HID = 256
H = 16
D = 16
HH = 128

NC = 2
NS = 16
NW = NC * NS

M1 = 80640
M2 = M - M1
G_CH = 128

S_PER_T = M // NS
S_FULL = S_PER_T // G_CH
S_REM = S_PER_T - S_FULL * G_CH

ROWS_PER_T = N // NS
PW = 144

R1 = 400
R2 = 640

_BF = jnp.bfloat16
_F = jnp.float32
_U = jnp.uint32


def _rne_hi(x):
    u = lax.bitcast_convert_type(x, _U)
    r = u + jnp.uint32(0x7FFF) + ((u >> jnp.uint32(16)) & jnp.uint32(1))
    return r & jnp.uint32(0xFFFF0000)


def _pack(a, b):
    return lax.bitcast_convert_type(_rne_hi(a) | (_rne_hi(b) >> jnp.uint32(16)), _F)


def _unpack(x):
    u = lax.bitcast_convert_type(x, _U)
    a = lax.bitcast_convert_type(u & jnp.uint32(0xFFFF0000), _F)
    b = lax.bitcast_convert_type(u << jnp.uint32(16), _F)
    return a, b


def _proj_body(xq_ref, xk_ref, wqa, wqb, wka, wkb, wva, wvb,
               q_ref, k_ref, v_ref):
    dn = (((1,), (1,)), ((), ()))
    xq = xq_ref[...]
    xk = xk_ref[...]
    q_ref[...] = _pack(lax.dot_general(xq, wqa[...], dn),
                       lax.dot_general(xq, wqb[...], dn))
    k_ref[...] = _pack(lax.dot_general(xk, wka[...], dn),
                       lax.dot_general(xk, wkb[...], dn))
    v_ref[...] = _pack(lax.dot_general(xk, wva[...], dn),
                       lax.dot_general(xk, wvb[...], dn))


def _proj_call(xq, xk, *ws):
    grid = (N // R1,)
    row = pl.BlockSpec((R1, HID), lambda i: (i, 0))
    half = pl.BlockSpec((R1, HH), lambda i: (i, 0))
    wfull = pl.BlockSpec((HH, HID), lambda i: (0, 0))
    return pl.pallas_call(
        _proj_body,
        grid=grid,
        in_specs=[row, row] + [wfull] * 6,
        out_specs=[half] * 3,
        out_shape=[jax.ShapeDtypeStruct((N, HH), _F)] * 3,
    )(xq, xk, *ws)


def _make_gather_body(base0, per_w):
    g_full = per_w // G_CH
    g_rem = per_w - g_full * G_CH

    def _gather_body(ta, tb, tc_, qi, ki, oa, ob, oc,
                     qi_t, ki_t, ra, rb, rc, sem_g, sem_w):
        c = lax.axis_index("c")
        s = lax.axis_index("s")
        w = s * NC + c
        base = base0 + w * per_w
        obase = w * per_w

        pltpu.sync_copy(qi.at[pl.ds(base, per_w)], qi_t)
        pltpu.sync_copy(ki.at[pl.ds(base, per_w)], ki_t)

        tabs = (ta, tb, tc_)
        outs = (oa, ob, oc)
        rows = (ra, rb, rc)
        idx_for = (0, 1, 1)

        def body(i, _):
            e0 = obase + i * G_CH
            o = i * G_CH
            sl = (qi_t.at[pl.ds(o, G_CH)], ki_t.at[pl.ds(o, G_CH)])
            cps = [pltpu.async_copy(t.at[sl[ix]], r, sem_g)
                   for t, r, ix in zip(tabs, rows, idx_for)]
            for cp in cps:
                cp.wait()
            wps = [pltpu.async_copy(r, ot.at[pl.ds(e0, G_CH)], sem_w)
                   for r, ot in zip(rows, outs)]
            for wp in wps:
                wp.wait()
            return _

        lax.fori_loop(0, g_full, body, 0)

        e0 = obase + g_full * G_CH
        o = g_full * G_CH
        sl = (qi_t.at[pl.ds(o, g_rem)], ki_t.at[pl.ds(o, g_rem)])
        cps = [pltpu.async_copy(t.at[sl[ix]], r.at[pl.ds(0, g_rem)], sem_g)
               for t, r, ix in zip(tabs, rows, idx_for)]
        for cp in cps:
            cp.wait()
        for r, ot in zip(rows, outs):
            pltpu.sync_copy(r.at[pl.ds(0, g_rem)], ot.at[pl.ds(e0, g_rem)])

    return _gather_body


def _gather_call(q, k, v, qi, ki, base0, mh):
    per_w = mh // NW
    mesh = plsc.VectorSubcoreMesh(core_axis_name="c", subcore_axis_name="s")
    f = pl.kernel(
        _make_gather_body(base0, per_w),
        mesh=mesh,
        compiler_params=pltpu.CompilerParams(use_tc_tiling_on_sc=False),
        out_type=[jax.ShapeDtypeStruct((mh, HH), _F)] * 3,
        scratch_types=[
            pltpu.VMEM((per_w,), jnp.int32),
            pltpu.VMEM((per_w,), jnp.int32),
            pltpu.VMEM((G_CH, HH), _F),
            pltpu.VMEM((G_CH, HH), _F),
            pltpu.VMEM((G_CH, HH), _F),
            pltpu.SemaphoreType.DMA,
            pltpu.SemaphoreType.DMA,
        ],
    )
    return f(q, k, v, qi, ki)


def _edge_body(qg_ref, kg_ref, vg_ref, ir_ref,
               wbma, wbmb, wbaa, wbab, s8_ref, st8_ref, inv_ref,
               logit_ref, ex_ref, numa_ref, numb_ref):
    dnt = (((1,), (1,)), ((), ()))
    dn = (((1,), (0,)), ((), ()))
    ir = ir_ref[...]
    bma = lax.dot_general(ir, wbma[...], dnt)
    bmb = lax.dot_general(ir, wbmb[...], dnt)
    baa = lax.dot_general(ir, wbaa[...], dnt)
    bab = lax.dot_general(ir, wbab[...], dnt)
    qga, qgb = _unpack(qg_ref[...])
    kga, kgb = _unpack(kg_ref[...])
    vga, vgb = _unpack(vg_ref[...])
    ta = kga * (1.0 + bma) + baa
    tb = kgb * (1.0 + bmb) + bab
    pa = qga * ta
    pb = qgb * tb
    s8 = s8_ref[...]
    inv = inv_ref[0, 0]
    la = lax.dot_general(pa, s8, dn) * inv
    lb = lax.dot_general(pb, s8, dn) * inv
    logit_ref[...] = jnp.concatenate([la, lb], axis=1)
    exa = jnp.exp(la)
    exb = jnp.exp(lb)
    z = jnp.zeros((la.shape[0], HH - 2 * 8), _F)
    ex_ref[...] = jnp.concatenate([exa, exb, z], axis=1)
    st8 = st8_ref[...]
    numa_ref[...] = lax.dot_general(exa, st8, dn) * vga
    numb_ref[...] = lax.dot_general(exb, st8, dn) * vgb


def _edge_call(qg, kg, vg, ir, wbma, wbmb, wbaa, wbab, s8, st8, inv,
               off_blocks, mh, carry):
    grid = (mh // R2,)
    off = off_blocks
    half = pl.BlockSpec((R2, HH), lambda i: (i, 0))
    ohalf = pl.BlockSpec((R2, HH), lambda i: (i + off, 0))
    wfull = pl.BlockSpec((HH, HID), lambda i: (0, 0))
    in_specs = ([half] * 3
                + [pl.BlockSpec((R2, HID), lambda i: (i + off, 0))]
                + [wfull] * 4
                + [pl.BlockSpec((HH, 8), lambda i: (0, 0)),
                   pl.BlockSpec((8, HH), lambda i: (0, 0)),
                   pl.BlockSpec(memory_space=pltpu.SMEM)])
    args = [qg, kg, vg, ir, wbma, wbmb, wbaa, wbab, s8, st8, inv]
    aliases = {}
    if carry is not None:
        in_specs = in_specs + [pl.BlockSpec(memory_space=pl.ANY)] * 4
        args = args + list(carry)
        aliases = {11: 0, 12: 1, 13: 2, 14: 3}

    def body(*refs):
        _edge_body(*refs[:11], *refs[-4:])

    return pl.pallas_call(
        body,
        grid=grid,
        in_specs=in_specs,
        out_specs=[pl.BlockSpec((R2, H), lambda i: (i + off, 0)),
                   ohalf, ohalf, ohalf],
        out_shape=[jax.ShapeDtypeStruct((M, H), _F),
                   jax.ShapeDtypeStruct((M, HH), _F),
                   jax.ShapeDtypeStruct((M, HH), _F),
                   jax.ShapeDtypeStruct((M, HH), _F)],
        input_output_aliases=aliases,
    )(*args)


def _scatter_body(numa, numb, ex, qi, z128, z16, out_o, accum,
                  idx_a, pay_a, idx_b, pay_b, idx_r, pay_r, sem_f):
    c = lax.axis_index("c")
    s = lax.axis_index("s")
    r0 = s * ROWS_PER_T
    col0 = c * PW
    exc0 = c * 8

    pltpu.sync_copy(z128.at[pl.ds(r0, ROWS_PER_T)],
                    accum.at[pl.ds(r0, ROWS_PER_T), pl.ds(0, HH)])
    pltpu.sync_copy(z16.at[pl.ds(r0, ROWS_PER_T)],
                    accum.at[pl.ds(r0, ROWS_PER_T), pl.ds(HH, 16)])
    plsc.subcore_barrier()

    def make_loop(num_ref):
        def body(i, _):
            e0 = s * S_PER_T + 2 * i * G_CH
            e1 = e0 + G_CH
            fa = [pltpu.async_copy(qi.at[pl.ds(e0, G_CH)], idx_a, sem_f),
                  pltpu.async_copy(num_ref.at[pl.ds(e0, G_CH)],
                                   pay_a.at[pl.ds(0, G_CH), pl.ds(0, HH)], sem_f),
                  pltpu.async_copy(ex.at[pl.ds(e0, G_CH), pl.ds(exc0, 8)],
                                   pay_a.at[pl.ds(0, G_CH), pl.ds(HH, 8)], sem_f)]
            fb = [pltpu.async_copy(qi.at[pl.ds(e1, G_CH)], idx_b, sem_f),
                  pltpu.async_copy(num_ref.at[pl.ds(e1, G_CH)],
                                   pay_b.at[pl.ds(0, G_CH), pl.ds(0, HH)], sem_f),
                  pltpu.async_copy(ex.at[pl.ds(e1, G_CH), pl.ds(exc0, 8)],
                                   pay_b.at[pl.ds(0, G_CH), pl.ds(HH, 8)], sem_f)]
            for f_ in fa:
                f_.wait()
            pltpu.sync_copy(pay_a, accum.at[idx_a], add=True)
            for f_ in fb:
                f_.wait()
            pltpu.sync_copy(pay_b, accum.at[idx_b], add=True)
            return _

        lax.fori_loop(0, S_FULL // 2, body, 0)
        e0 = s * S_PER_T + S_FULL * G_CH
        pltpu.sync_copy(qi.at[pl.ds(e0, S_REM)], idx_r)
        pltpu.sync_copy(num_ref.at[pl.ds(e0, S_REM)],
                        pay_r.at[pl.ds(0, S_REM), pl.ds(0, HH)])
        pltpu.sync_copy(ex.at[pl.ds(e0, S_REM), pl.ds(exc0, 8)],
                        pay_r.at[pl.ds(0, S_REM), pl.ds(HH, 8)])
        pltpu.sync_copy(pay_r, accum.at[idx_r], add=True)

    @pl.when(c == 0)
    def _():
        make_loop(numa)

    @pl.when(c != 0)
    def _():
        make_loop(numb)

    plsc.subcore_barrier()
    pltpu.sync_copy(accum.at[pl.ds(r0, ROWS_PER_T)],
                    out_o.at[pl.ds(r0, ROWS_PER_T), pl.ds(col0, PW)])


def _scatter_call(numa, numb, ex, qi, z128, z16):
    mesh = plsc.VectorSubcoreMesh(core_axis_name="c", subcore_axis_name="s")
    f = pl.kernel(
        _scatter_body,
        mesh=mesh,
        compiler_params=pltpu.CompilerParams(use_tc_tiling_on_sc=False),
        out_type=jax.ShapeDtypeStruct((N, 2 * PW), _F),
        scratch_types=[
            pltpu.VMEM_SHARED((N, PW), _F),
            pltpu.VMEM((G_CH,), jnp.int32),
            pltpu.VMEM((G_CH, PW), _F),
            pltpu.VMEM((G_CH,), jnp.int32),
            pltpu.VMEM((G_CH, PW), _F),
            pltpu.VMEM((S_REM,), jnp.int32),
            pltpu.VMEM((S_REM, PW), _F),
            pltpu.SemaphoreType.DMA,
        ],
    )
    return f(numa, numb, ex, qi, z128, z16)


def _final_body(acc_ref, st8_ref, wo_ref, res_ref):
    a = acc_ref[...]
    dn = (((1,), (0,)), ((), ()))
    st8 = st8_ref[...]
    den_a = lax.dot_general(a[:, 128:136], st8, dn) + 1e-16
    den_b = lax.dot_general(a[:, 272:280], st8, dn) + 1e-16
    r = jnp.concatenate([a[:, 0:128] / den_a, a[:, 144:272] / den_b], axis=1)
    res_ref[...] = lax.dot_general(r, wo_ref[...], (((1,), (1,)), ((), ())))


def _final_call(acc, st8, wo):
    grid = (N // R1,)
    return pl.pallas_call(
        _final_body,
        grid=grid,
        in_specs=[pl.BlockSpec((R1, 2 * PW), lambda i: (i, 0)),
                  pl.BlockSpec((8, HH), lambda i: (0, 0)),
                  pl.BlockSpec((HID, HID), lambda i: (0, 0))],
        out_specs=pl.BlockSpec((R1, HID), lambda i: (i, 0)),
        out_shape=jax.ShapeDtypeStruct((N, HID), _F),
    )(acc, st8, wo)


def _head_major_rows(w):
    return w.reshape(D, H, HID).transpose(1, 0, 2).reshape(H * D, HID)


def kernel(query, key, query_index, key_index, interaction_repr,
           Wq, Wkv, Wb, Wo, normalizer):
    wq = _head_major_rows(Wq)
    wk = _head_major_rows(Wkv[:HID])
    wv = _head_major_rows(Wkv[HID:])
    wbm = _head_major_rows(Wb[:HID])
    wba = _head_major_rows(Wb[HID:])
    wo = Wo.reshape(HID, D, H).transpose(0, 2, 1).reshape(HID, HID)

    inv = (1.0 / jnp.clip(normalizer, 1.0, float(D))).reshape(1, 1)
    s8 = jnp.repeat(jnp.eye(8, dtype=_F), D, axis=0)
    st8 = jnp.repeat(jnp.eye(8, dtype=_F), D, axis=1)
    z128 = jnp.zeros((N, HH), _F)
    z16 = jnp.zeros((N, 16), _F)

    q, k, v = _proj_call(
        query, key, wq[:HH], wq[HH:], wk[:HH], wk[HH:], wv[:HH], wv[HH:])
    wb_args = (wbm[:HH], wbm[HH:], wba[:HH], wba[HH:], s8, st8, inv)
    qg1, kg1, vg1 = _gather_call(q, k, v, query_index, key_index, 0, M1)
    qg2, kg2, vg2 = _gather_call(q, k, v, query_index, key_index, M1, M2)
    carry = _edge_call(qg1, kg1, vg1, interaction_repr, *wb_args,
                       0, M1, None)
    logits, ex, numa, numb = _edge_call(qg2, kg2, vg2, interaction_repr,
                                        *wb_args, M1 // R2, M2, carry)
    acc = _scatter_call(numa, numb, ex, query_index, z128, z16)
    result = _final_call(acc, st8, wo)
    return (result, logits)
